# baseline (speedup 1.0000x reference)
import dataclasses

import jax
import jax.numpy as jnp
from jax import lax
from jax.experimental import pallas as pl
from jax.experimental.pallas import tpu as pltpu
from jax.experimental.pallas import tpu_sc as plsc

_C = 16
_B = 16384
_NW = 32
_EPW = _B // _NW
_PITCH = 17

_LOG1P_P = (-0.008466129016818367, 0.0436580512857575, -0.10679717589934831,
            0.1765967880850523, -0.24453302495503662, 0.3326523501519017,
            -0.49996354303547863, 0.9999995178202268)


def _sc_body(lbl_hbm, log_hbm, dom_hbm, tsi_hbm, out_hbm,
             lblv, lblp, idxv, xv, dv,
             tA_t, cA_t, tB_t, cB_t,
             oA, oCA, oB, oCB, tsis):
    wid = lax.axis_index("s") * 2 + lax.axis_index("c")
    base = wid * _EPW

    pltpu.sync_copy(tsi_hbm, tsis)
    pltpu.sync_copy(lbl_hbm.at[pl.ds(base * _C, _EPW * _C)], lblv)
    pltpu.sync_copy(dom_hbm.at[pl.ds(base, _EPW)], dv)

    iota = lax.broadcasted_iota(jnp.int32, (_C,), 0)
    zero16 = jnp.zeros((_C,), jnp.float32)
    ones16 = jnp.ones((_C,), jnp.float32)

    for j in range(_PITCH):
        tA_t[pl.ds(j * _C, _C)] = zero16
        cA_t[pl.ds(j * _C, _C)] = zero16
        tB_t[pl.ds(j * _C, _C)] = zero16
        cB_t[pl.ds(j * _C, _C)] = zero16

    @pl.loop(0, _EPW)
    def _(j):
        row = lblv[pl.ds(j * _C, _C)]
        plsc.store_scatter(lblp, [j * _PITCH + iota], row)

    @pl.loop(0, _EPW // _C)
    def _(k):
        rowb = (k * _C + iota) * _PITCH
        best = plsc.load_gather(lblp, [rowb])
        bidx = jnp.zeros((_C,), jnp.int32)
        for c in range(1, _C):
            v = plsc.load_gather(lblp, [rowb + c])
            gt = v > best
            best = jnp.where(gt, v, best)
            bidx = jnp.where(gt, c, bidx)
        g = base + k * _C + iota
        idxv[pl.ds(k * _C, _C)] = bidx * _B + g

    pltpu.sync_copy(log_hbm.at[idxv], xv)

    tsiv = tsis[...]

    @pl.loop(0, _EPW // _C)
    def _(k):
        sl = pl.ds(k * _C, _C)
        x = xv[sl]
        d = dv[sl]
        cls = lax.shift_right_logical(idxv[sl], 14)
        u = jnp.exp(-jnp.abs(x))
        p = jnp.full((_C,), _LOG1P_P[0], jnp.float32)
        for coef in _LOG1P_P[1:]:
            p = p * u + coef
        sp = u * p
        tA = jnp.maximum(x, 0.0) - x * d + sp
        tB = tA + x * (2.0 * d - 1.0)
        g = base + k * _C + iota
        m = jnp.where(g >= tsiv, 1.0, 0.0).astype(jnp.float32)
        addr = iota * _PITCH + cls
        plsc.addupdate_scatter(tA_t, [addr], tA)
        plsc.addupdate_scatter(cA_t, [addr], ones16)
        plsc.addupdate_scatter(tB_t, [addr], tB * m)
        plsc.addupdate_scatter(cB_t, [addr], m)

    accA = zero16
    accCA = zero16
    accB = zero16
    accCB = zero16
    for j in range(_C):
        adr = j * _PITCH + iota
        accA = accA + plsc.load_gather(tA_t, [adr])
        accCA = accCA + plsc.load_gather(cA_t, [adr])
        accB = accB + plsc.load_gather(tB_t, [adr])
        accCB = accCB + plsc.load_gather(cB_t, [adr])
    oA[...] = accA
    oCA[...] = accCA
    oB[...] = accB
    oCB[...] = accCB
    pltpu.sync_copy(oA, out_hbm.at[0, wid])
    pltpu.sync_copy(oCA, out_hbm.at[1, wid])
    pltpu.sync_copy(oB, out_hbm.at[2, wid])
    pltpu.sync_copy(oCB, out_hbm.at[3, wid])


def _fin_body(p_ref, outA_ref, outB_ref):
    s = jnp.sum(p_ref[...], axis=1)
    lossA = jnp.sum(s[0] / s[1]) * (1.0 / _C)
    lossB = jnp.sum(s[2] / s[3]) * (1.0 / _C)
    outA_ref[...] = jnp.broadcast_to(lossA, (1, 1))
    outB_ref[...] = jnp.broadcast_to(lossB, (1, 1))


def kernel(logits_list, labels, domain, target_start_id):
    lbl_flat = labels.reshape(-1)
    log_flat = logits_list.reshape(-1)
    dom_flat = domain.reshape(-1)
    tsi = jnp.broadcast_to(jnp.asarray(target_start_id, jnp.int32), (_C,))

    mesh = plsc.VectorSubcoreMesh(core_axis_name="c", subcore_axis_name="s")
    cp = pltpu.CompilerParams()
    if "needs_layout_passes" in pltpu.CompilerParams.__dataclass_fields__:
        cp = dataclasses.replace(cp, needs_layout_passes=False)
    sc_call = pl.kernel(
        out_type=jax.ShapeDtypeStruct((4, _NW, _C), jnp.float32),
        mesh=mesh,
        compiler_params=cp,
        scratch_types=[
            pltpu.VMEM((_EPW * _C,), jnp.float32),
            pltpu.VMEM((_EPW * _PITCH,), jnp.float32),
            pltpu.VMEM((_EPW,), jnp.int32),
            pltpu.VMEM((_EPW,), jnp.float32),
            pltpu.VMEM((_EPW,), jnp.float32),
            pltpu.VMEM((_C * _PITCH,), jnp.float32),
            pltpu.VMEM((_C * _PITCH,), jnp.float32),
            pltpu.VMEM((_C * _PITCH,), jnp.float32),
            pltpu.VMEM((_C * _PITCH,), jnp.float32),
            pltpu.VMEM((_C,), jnp.float32),
            pltpu.VMEM((_C,), jnp.float32),
            pltpu.VMEM((_C,), jnp.float32),
            pltpu.VMEM((_C,), jnp.float32),
            pltpu.VMEM((_C,), jnp.int32),
        ],
    )(_sc_body)
    partials = sc_call(lbl_flat, log_flat, dom_flat, tsi)

    outA, outB = pl.pallas_call(
        _fin_body,
        out_shape=(jax.ShapeDtypeStruct((1, 1), jnp.float32),
                   jax.ShapeDtypeStruct((1, 1), jnp.float32)),
    )(partials)
    return (outA[0, 0], outB[0, 0])


# speedup vs baseline: 4.2070x; 4.2070x over previous
import jax
import jax.numpy as jnp
from jax.experimental import pallas as pl
from jax.experimental.pallas import tpu as pltpu

_C = 16
_R = 128
_G = 8
_RB = _R // _G


def _loss_body(tsi_ref, logits_ref, labelsT_ref, domain_ref,
               outA_ref, outB_ref, sA, cA, sB, cB):
    i = pl.program_id(0)

    @pl.when(i == 0)
    def _():
        z = jnp.zeros((_C,), jnp.float32)
        sA[...] = z
        cA[...] = z
        sB[...] = z
        cB[...] = z

    lbl = labelsT_ref[...]
    ci = jax.lax.broadcasted_iota(jnp.int32, (_C, _RB, _R), 0)
    mx = jnp.max(lbl, axis=0, keepdims=True)
    cls = jnp.min(jnp.where(lbl == mx, ci, _C), axis=0, keepdims=True)
    onehot = (ci == cls).astype(jnp.float32)

    x = jnp.sum(logits_ref[...] * onehot, axis=0)
    d = domain_ref[...]
    sp = jnp.log1p(jnp.exp(-jnp.abs(x)))
    tA = jnp.maximum(x, 0.0) - x * d + sp
    tB = tA + x * (2.0 * d - 1.0)

    bidx = ((i * _RB + jax.lax.broadcasted_iota(jnp.int32, (_RB, _R), 0)) * _R
            + jax.lax.broadcasted_iota(jnp.int32, (_RB, _R), 1))
    tgt = (bidx >= tsi_ref[0]).astype(jnp.float32)

    oh_tgt = onehot * tgt[None]
    sA[...] += jnp.sum(onehot * tA[None], axis=(1, 2))
    cA[...] += jnp.sum(onehot, axis=(1, 2))
    sB[...] += jnp.sum(oh_tgt * tB[None], axis=(1, 2))
    cB[...] += jnp.sum(oh_tgt, axis=(1, 2))

    @pl.when(i == _G - 1)
    def _():
        lossA = jnp.sum(sA[...] / cA[...]) * (1.0 / _C)
        lossB = jnp.sum(sB[...] / cB[...]) * (1.0 / _C)
        outA_ref[...] = jnp.broadcast_to(lossA, (1, 1))
        outB_ref[...] = jnp.broadcast_to(lossB, (1, 1))


def kernel(logits_list, labels, domain, target_start_id):
    logits3 = logits_list.reshape(_C, _R, _R)
    dom = domain.reshape(_R, _R)
    tsi = jnp.asarray(target_start_id, jnp.int32).reshape(1)

    outA, outB = pl.pallas_call(
        _loss_body,
        grid=(_G,),
        out_shape=(jax.ShapeDtypeStruct((1, 1), jnp.float32),
                   jax.ShapeDtypeStruct((1, 1), jnp.float32)),
        in_specs=[
            pl.BlockSpec(memory_space=pltpu.SMEM),
            pl.BlockSpec((_C, _RB, _R), lambda i: (0, i, 0)),
            pl.BlockSpec((_C, _RB, _R), lambda i: (0, i, 0)),
            pl.BlockSpec((_RB, _R), lambda i: (i, 0)),
        ],
        out_specs=(pl.BlockSpec((1, 1), lambda i: (0, 0)),
                   pl.BlockSpec((1, 1), lambda i: (0, 0))),
        scratch_shapes=[pltpu.VMEM((_C,), jnp.float32)] * 4,
    )(tsi, logits3, labels.T.reshape(_C, _R, _R), dom)
    return (outA[0, 0], outB[0, 0])


# speedup vs baseline: 6.3057x; 1.4989x over previous
import jax
import jax.numpy as jnp
from jax.experimental import pallas as pl
from jax.experimental.pallas import tpu as pltpu

_C = 16
_R = 128


def _loss_body(tsi_ref, logits_ref, labels_ref, domain_ref, outA_ref, outB_ref):
    lbl = labels_ref[...]
    ci = jax.lax.broadcasted_iota(jnp.int32, (_C, _R, _R), 0)
    mx = jnp.max(lbl, axis=0, keepdims=True)
    cls = jnp.min(jnp.where(lbl == mx, ci, _C), axis=0, keepdims=True)
    onehot = (ci == cls).astype(jnp.float32)

    x = jnp.sum(logits_ref[...] * onehot, axis=0)
    d = domain_ref[...]
    sp = jnp.log1p(jnp.exp(-jnp.abs(x)))
    tA = jnp.maximum(x, 0.0) - x * d + sp
    tB = tA + x * (2.0 * d - 1.0)

    bidx = (jax.lax.broadcasted_iota(jnp.int32, (_R, _R), 0) * _R
            + jax.lax.broadcasted_iota(jnp.int32, (_R, _R), 1))
    tgt = (bidx >= tsi_ref[0]).astype(jnp.float32)

    sumA = jnp.sum(onehot * tA[None], axis=(1, 2))
    cntA = jnp.sum(onehot, axis=(1, 2))
    oh_tgt = onehot * tgt[None]
    sumB = jnp.sum(oh_tgt * tB[None], axis=(1, 2))
    cntB = jnp.sum(oh_tgt, axis=(1, 2))

    lossA = jnp.sum(sumA / cntA) * (1.0 / _C)
    lossB = jnp.sum(sumB / cntB) * (1.0 / _C)
    outA_ref[...] = jnp.broadcast_to(lossA, (1, 1))
    outB_ref[...] = jnp.broadcast_to(lossB, (1, 1))


def kernel(logits_list, labels, domain, target_start_id):
    logits3 = logits_list.reshape(_C, _R, _R)
    dom = domain.reshape(_R, _R)
    tsi = jnp.asarray(target_start_id, jnp.int32).reshape(1)

    outA, outB = pl.pallas_call(
        _loss_body,
        out_shape=(jax.ShapeDtypeStruct((1, 1), jnp.float32),
                   jax.ShapeDtypeStruct((1, 1), jnp.float32)),
        in_specs=[
            pl.BlockSpec(memory_space=pltpu.SMEM),
            pl.BlockSpec(memory_space=pltpu.VMEM),
            pl.BlockSpec(memory_space=pltpu.VMEM),
            pl.BlockSpec(memory_space=pltpu.VMEM),
        ],
    )(tsi, logits3, labels.T.reshape(_C, _R, _R), dom)
    return (outA[0, 0], outB[0, 0])
